# Initial kernel scaffold; baseline (speedup 1.0000x reference)
import jax
import jax.numpy as jnp
from jax.experimental import pallas as pl


def kernel(x, table, W1, b1, W2, b2, W3, b3):
    raise NotImplementedError("write your pallas kernel here")


# speedup vs baseline: 7.8747x; 7.8747x over previous
import functools

import jax
import jax.numpy as jnp
from jax import lax
from jax.experimental import pallas as pl
from jax.experimental.pallas import tpu as pltpu
from jax.experimental.pallas import tpu_sc as plsc

_B, _L, _D = 16384, 200, 32
_H = 150
_NC, _NS = 2, 16
_NW = _NC * _NS
_EPW = _B // _NW
_CH = 64
_HL = _L // 2


def _pool_sc(x3, table):
  mesh = plsc.VectorSubcoreMesh(core_axis_name="c", subcore_axis_name="s")

  @functools.partial(
      pl.kernel,
      mesh=mesh,
      compiler_params=pltpu.CompilerParams(use_tc_tiling_on_sc=False),
      out_type=jax.ShapeDtypeStruct((_B, _D), jnp.float32),
      scratch_types=[
          pltpu.VMEM((_CH, 2, _HL), jnp.int32),
          pltpu.VMEM((2, _HL, _D), jnp.float32),
          pltpu.VMEM((_CH, _D), jnp.float32),
          pltpu.SemaphoreType.DMA,
      ],
  )
  def body(x_hbm, tab_hbm, out_hbm, idx_v, rows_v, pool_v, sem):
    wid = lax.axis_index("s") * _NC + lax.axis_index("c")
    base = wid * _EPW

    def chunk_body(ci, carry):
      cbase = base + ci * _CH
      pltpu.sync_copy(x_hbm.at[pl.ds(cbase, _CH)], idx_v)

      def elem_body(j, carry2):
        pltpu.async_copy(tab_hbm.at[idx_v.at[j, 0]], rows_v.at[0], sem).wait()
        pltpu.async_copy(tab_hbm.at[idx_v.at[j, 1]], rows_v.at[1], sem).wait()

        def row_body(r, accs):
          a0, a1 = accs
          a0 = a0 + rows_v[0, r, pl.ds(0, 16)] + rows_v[1, r, pl.ds(0, 16)]
          a1 = a1 + rows_v[0, r, pl.ds(16, 16)] + rows_v[1, r, pl.ds(16, 16)]
          return (a0, a1)

        z = jnp.zeros((16,), jnp.float32)
        a0, a1 = lax.fori_loop(0, _HL, row_body, (z, z))
        pool_v[j, pl.ds(0, 16)] = a0 * (1.0 / _L)
        pool_v[j, pl.ds(16, 16)] = a1 * (1.0 / _L)
        return carry2

      lax.fori_loop(0, _CH, elem_body, 0)
      pltpu.sync_copy(pool_v, out_hbm.at[pl.ds(cbase, _CH)])
      return carry

    lax.fori_loop(0, _EPW // _CH, chunk_body, 0)

  return body(x3, table)


def _mlp_tc(pooled, W1, b1, W2, b2, W3, b3):
  bb = 2048

  def body(p_ref, w1_ref, b1_ref, w2_ref, b2_ref, w3_ref, b3_ref, o_ref):
    h = jnp.dot(p_ref[...], w1_ref[...], preferred_element_type=jnp.float32)
    h = jnp.maximum(h + b1_ref[...], 0.0)
    h = jnp.dot(h, w2_ref[...], preferred_element_type=jnp.float32)
    h = jnp.maximum(h + b2_ref[...], 0.0)
    o_ref[...] = (
        jnp.dot(h, w3_ref[...], preferred_element_type=jnp.float32)
        + b3_ref[...]
    )

  return pl.pallas_call(
      body,
      grid=(_B // bb,),
      in_specs=[
          pl.BlockSpec((bb, _D), lambda i: (i, 0)),
          pl.BlockSpec((_D, _H), lambda i: (0, 0)),
          pl.BlockSpec((1, _H), lambda i: (0, 0)),
          pl.BlockSpec((_H, _H), lambda i: (0, 0)),
          pl.BlockSpec((1, _H), lambda i: (0, 0)),
          pl.BlockSpec((_H, 1), lambda i: (0, 0)),
          pl.BlockSpec((1, 1), lambda i: (0, 0)),
      ],
      out_specs=pl.BlockSpec((bb, 1), lambda i: (i, 0)),
      out_shape=jax.ShapeDtypeStruct((_B, 1), jnp.float32),
  )(pooled, W1, b1.reshape(1, _H), W2, b2.reshape(1, _H), W3,
    b3.reshape(1, 1))


@jax.jit
def _run(x, table, W1, b1, W2, b2, W3, b3):
  x3 = x.reshape(_B, 2, _HL).astype(jnp.int32)
  pooled = _pool_sc(x3, table)
  return _mlp_tc(pooled, W1, b1, W2, b2, W3, b3)


def kernel(x, table, W1, b1, W2, b2, W3, b3):
  return _run(x, table, W1, b1, W2, b2, W3, b3)


# speedup vs baseline: 13.1078x; 1.6645x over previous
import functools

import jax
import jax.numpy as jnp
from jax import lax
from jax.experimental import pallas as pl
from jax.experimental.pallas import tpu as pltpu
from jax.experimental.pallas import tpu_sc as plsc

_B, _L, _D = 16384, 200, 32
_H = 150
_NC, _NS = 2, 16
_NW = _NC * _NS
_EPW = _B // _NW
_CH = 64
_HL = _L // 2


def _pool_sc(x3, table):
  mesh = plsc.VectorSubcoreMesh(core_axis_name="c", subcore_axis_name="s")

  @functools.partial(
      pl.kernel,
      mesh=mesh,
      compiler_params=pltpu.CompilerParams(use_tc_tiling_on_sc=False),
      out_type=jax.ShapeDtypeStruct((_B, _D), jnp.float32),
      scratch_types=[
          pltpu.VMEM((_CH, 2, _HL), jnp.int32),
          pltpu.VMEM((2, 2, _HL, _D), jnp.float32),
          pltpu.VMEM((_CH, _D), jnp.float32),
          pltpu.SemaphoreType.DMA,
          pltpu.SemaphoreType.DMA,
      ],
  )
  def body(x_hbm, tab_hbm, out_hbm, idx_v, rows_v, pool_v, sem_a, sem_b):
    wid = lax.axis_index("s") * _NC + lax.axis_index("c")
    base = wid * _EPW

    def fire(j, buf, sem):
      pltpu.async_copy(tab_hbm.at[idx_v.at[j, 0]], rows_v.at[buf, 0], sem)
      pltpu.async_copy(tab_hbm.at[idx_v.at[j, 1]], rows_v.at[buf, 1], sem)

    def drain(buf, sem):
      pltpu.make_async_copy(
          tab_hbm.at[idx_v.at[0, 0]], rows_v.at[buf, 0], sem).wait()
      pltpu.make_async_copy(
          tab_hbm.at[idx_v.at[0, 1]], rows_v.at[buf, 1], sem).wait()

    def accum(j, buf):
      def row_body(r, accs):
        a0, a1 = accs
        a0 = a0 + rows_v[buf, 0, r, pl.ds(0, 16)]
        a0 = a0 + rows_v[buf, 1, r, pl.ds(0, 16)]
        a1 = a1 + rows_v[buf, 0, r, pl.ds(16, 16)]
        a1 = a1 + rows_v[buf, 1, r, pl.ds(16, 16)]
        return (a0, a1)

      z = jnp.zeros((16,), jnp.float32)
      a0, a1 = lax.fori_loop(0, _HL, row_body, (z, z))
      pool_v[j, pl.ds(0, 16)] = a0 * (1.0 / _L)
      pool_v[j, pl.ds(16, 16)] = a1 * (1.0 / _L)

    def chunk_body(ci, carry):
      cbase = base + ci * _CH
      pltpu.sync_copy(x_hbm.at[pl.ds(cbase, _CH)], idx_v)
      fire(0, 0, sem_a)

      def pair_body(p, carry2):
        j0 = 2 * p
        fire(j0 + 1, 1, sem_b)
        drain(0, sem_a)
        accum(j0, 0)

        @pl.when(j0 + 2 < _CH)
        def _():
          fire(j0 + 2, 0, sem_a)

        drain(1, sem_b)
        accum(j0 + 1, 1)
        return carry2

      lax.fori_loop(0, _CH // 2, pair_body, 0)
      pltpu.sync_copy(pool_v, out_hbm.at[pl.ds(cbase, _CH)])
      return carry

    lax.fori_loop(0, _EPW // _CH, chunk_body, 0)

  return body(x3, table)


def _mlp_tc(pooled, W1, b1, W2, b2, W3, b3):
  bb = 2048

  def body(p_ref, w1_ref, b1_ref, w2_ref, b2_ref, w3_ref, b3_ref, o_ref):
    h = jnp.dot(p_ref[...], w1_ref[...], preferred_element_type=jnp.float32)
    h = jnp.maximum(h + b1_ref[...], 0.0)
    h = jnp.dot(h, w2_ref[...], preferred_element_type=jnp.float32)
    h = jnp.maximum(h + b2_ref[...], 0.0)
    o_ref[...] = (
        jnp.dot(h, w3_ref[...], preferred_element_type=jnp.float32)
        + b3_ref[...]
    )

  return pl.pallas_call(
      body,
      grid=(_B // bb,),
      in_specs=[
          pl.BlockSpec((bb, _D), lambda i: (i, 0)),
          pl.BlockSpec((_D, _H), lambda i: (0, 0)),
          pl.BlockSpec((1, _H), lambda i: (0, 0)),
          pl.BlockSpec((_H, _H), lambda i: (0, 0)),
          pl.BlockSpec((1, _H), lambda i: (0, 0)),
          pl.BlockSpec((_H, 1), lambda i: (0, 0)),
          pl.BlockSpec((1, 1), lambda i: (0, 0)),
      ],
      out_specs=pl.BlockSpec((bb, 1), lambda i: (i, 0)),
      out_shape=jax.ShapeDtypeStruct((_B, 1), jnp.float32),
  )(pooled, W1, b1.reshape(1, _H), W2, b2.reshape(1, _H), W3,
    b3.reshape(1, 1))


@jax.jit
def _run(x, table, W1, b1, W2, b2, W3, b3):
  x3 = x.reshape(_B, 2, _HL).astype(jnp.int32)
  pooled = _pool_sc(x3, table)
  return _mlp_tc(pooled, W1, b1, W2, b2, W3, b3)


def kernel(x, table, W1, b1, W2, b2, W3, b3):
  return _run(x, table, W1, b1, W2, b2, W3, b3)
